# baseline (speedup 1.0000x reference)
import functools
import math

import jax
import jax.numpy as jnp
from jax import lax
from jax.experimental import pallas as pl
from jax.experimental.pallas import tpu as pltpu
from jax.experimental.pallas import tpu_sc as plsc

S = 64.0
M2 = 0.5
THETA = math.cos(math.pi - M2)
COS_M2 = math.cos(M2)
SIN_M2 = math.sin(M2)
EPS = 1e-07

NC = 2
NS = 16
LANES = 16

BR = 256
BC = 2048


def _sqrt16(a):
    seed_bits = jnp.int32(0x1FBD1DF5) + (lax.bitcast_convert_type(a, jnp.int32) >> 1)
    y = lax.bitcast_convert_type(seed_bits, jnp.float32)
    for _ in range(3):
        y = 0.5 * (y + a / y)
    return y


def _sc_body(n_cols, labels_hbm, logits_hbm, out_hbm, lab_v, idx_v, val_v, res_v, sem):
    wid = lax.axis_index("s") * NC + lax.axis_index("c")
    bpw = 2 * LANES
    base = wid * bpw
    pltpu.sync_copy(labels_hbm.at[pl.ds(base, bpw)], lab_v)
    for j in range(2):
        lab16 = lab_v[pl.ds(j * LANES, LANES)]
        rows = lax.iota(jnp.int32, LANES) + (base + j * LANES)
        idx_v[pl.ds(j * LANES, LANES)] = rows * n_cols + lab16
    pltpu.async_copy(logits_hbm.at[idx_v], val_v, sem).wait()
    for j in range(2):
        x = val_v[pl.ds(j * LANES, LANES)]
        tlc = jnp.clip(x, -1.0 + EPS, 1.0 - EPS)
        root = _sqrt16(jnp.maximum(1.0 - tlc * tlc, 1e-30))
        margined = tlc * COS_M2 - root * SIN_M2
        newv = jnp.where(x > THETA, margined, tlc)
        res_v[pl.ds(j * LANES, LANES)] = newv * S
    pltpu.sync_copy(res_v, out_hbm.at[pl.ds(base, bpw)])


def _sc_new_targets(labels, logits_flat, n_cols):
    n_rows = labels.shape[0]
    mesh = plsc.VectorSubcoreMesh(core_axis_name="c", subcore_axis_name="s")
    return pl.kernel(
        functools.partial(_sc_body, n_cols),
        out_type=jax.ShapeDtypeStruct((n_rows,), jnp.float32),
        mesh=mesh,
        scratch_types=[
            pltpu.VMEM((32,), jnp.int32),
            pltpu.VMEM((32,), jnp.int32),
            pltpu.VMEM((32,), jnp.float32),
            pltpu.VMEM((32,), jnp.float32),
            pltpu.SemaphoreType.DMA,
        ],
    )(labels, logits_flat)


def _tc_body(lab_ref, new_ref, x_ref, o_ref):
    j = pl.program_id(1)
    lab = lab_ref[...]
    nv = new_ref[...]
    local = lab - j * BC
    col_ids = jax.lax.broadcasted_iota(jnp.int32, (BR, BC), 1)
    mask = col_ids == local
    o_ref[...] = jnp.where(mask, nv, x_ref[...] * S)


def kernel(logits, labels):
    n_rows, n_cols = logits.shape
    newv = _sc_new_targets(labels, logits.reshape(-1), n_cols)
    grid = (n_rows // BR, pl.cdiv(n_cols, BC))
    return pl.pallas_call(
        _tc_body,
        grid=grid,
        in_specs=[
            pl.BlockSpec((BR, 1), lambda i, j: (i, 0)),
            pl.BlockSpec((BR, 1), lambda i, j: (i, 0)),
            pl.BlockSpec((BR, BC), lambda i, j: (i, j)),
        ],
        out_specs=pl.BlockSpec((BR, BC), lambda i, j: (i, j)),
        out_shape=jax.ShapeDtypeStruct((n_rows, n_cols), logits.dtype),
        compiler_params=pltpu.CompilerParams(
            dimension_semantics=("parallel", "parallel"),
        ),
    )(labels.reshape(n_rows, 1), newv.reshape(n_rows, 1), logits)


# speedup vs baseline: 1.0151x; 1.0151x over previous
import functools
import math

import jax
import jax.numpy as jnp
from jax import lax
from jax.experimental import pallas as pl
from jax.experimental.pallas import tpu as pltpu
from jax.experimental.pallas import tpu_sc as plsc

S = 64.0
M2 = 0.5
THETA = math.cos(math.pi - M2)
COS_M2 = math.cos(M2)
SIN_M2 = math.sin(M2)
EPS = 1e-07

NC = 2
NS = 16
LANES = 16

BR = 16
BC = 100000


def _sqrt16(a):
    seed_bits = jnp.int32(0x1FBD1DF5) + (lax.bitcast_convert_type(a, jnp.int32) >> 1)
    y = lax.bitcast_convert_type(seed_bits, jnp.float32)
    for _ in range(3):
        y = 0.5 * (y + a / y)
    return y


def _sc_body(n_cols, labels_hbm, logits_hbm, out_hbm, lab_v, idx_v, val_v, res_v, sem):
    wid = lax.axis_index("s") * NC + lax.axis_index("c")
    bpw = 2 * LANES
    base = wid * bpw
    pltpu.sync_copy(labels_hbm.at[pl.ds(base, bpw)], lab_v)
    for j in range(2):
        lab16 = lab_v[pl.ds(j * LANES, LANES)]
        rows = lax.iota(jnp.int32, LANES) + (base + j * LANES)
        idx_v[pl.ds(j * LANES, LANES)] = rows * n_cols + lab16
    pltpu.async_copy(logits_hbm.at[idx_v], val_v, sem).wait()
    for j in range(2):
        x = val_v[pl.ds(j * LANES, LANES)]
        tlc = jnp.clip(x, -1.0 + EPS, 1.0 - EPS)
        root = _sqrt16(jnp.maximum(1.0 - tlc * tlc, 1e-30))
        margined = tlc * COS_M2 - root * SIN_M2
        newv = jnp.where(x > THETA, margined, tlc)
        res_v[pl.ds(j * LANES, LANES)] = newv * S
    pltpu.sync_copy(res_v, out_hbm.at[pl.ds(base, bpw)])


def _sc_new_targets(labels, logits_flat, n_cols):
    n_rows = labels.shape[0]
    mesh = plsc.VectorSubcoreMesh(core_axis_name="c", subcore_axis_name="s")
    return pl.kernel(
        functools.partial(_sc_body, n_cols),
        out_type=jax.ShapeDtypeStruct((n_rows,), jnp.float32),
        mesh=mesh,
        scratch_types=[
            pltpu.VMEM((32,), jnp.int32),
            pltpu.VMEM((32,), jnp.int32),
            pltpu.VMEM((32,), jnp.float32),
            pltpu.VMEM((32,), jnp.float32),
            pltpu.SemaphoreType.DMA,
        ],
    )(labels, logits_flat)


def _tc_body(lab_ref, new_ref, x_ref, o_ref):
    j = pl.program_id(1)
    lab = lab_ref[...]
    nv = new_ref[...]
    local = lab - j * BC
    col_ids = jax.lax.broadcasted_iota(jnp.int32, (BR, BC), 1)
    mask = col_ids == local
    o_ref[...] = jnp.where(mask, nv, x_ref[...] * S)


def kernel(logits, labels):
    n_rows, n_cols = logits.shape
    newv = _sc_new_targets(labels, logits.reshape(-1), n_cols)
    grid = (n_rows // BR, pl.cdiv(n_cols, BC))
    return pl.pallas_call(
        _tc_body,
        grid=grid,
        in_specs=[
            pl.BlockSpec((BR, 1), lambda i, j: (i, 0)),
            pl.BlockSpec((BR, 1), lambda i, j: (i, 0)),
            pl.BlockSpec((BR, BC), lambda i, j: (i, j)),
        ],
        out_specs=pl.BlockSpec((BR, BC), lambda i, j: (i, j)),
        out_shape=jax.ShapeDtypeStruct((n_rows, n_cols), logits.dtype),
        compiler_params=pltpu.CompilerParams(
            dimension_semantics=("parallel", "parallel"),
        ),
    )(labels.reshape(n_rows, 1), newv.reshape(n_rows, 1), logits)


# speedup vs baseline: 1.0176x; 1.0024x over previous
import functools
import math

import jax
import jax.numpy as jnp
from jax import lax
from jax.experimental import pallas as pl
from jax.experimental.pallas import tpu as pltpu
from jax.experimental.pallas import tpu_sc as plsc

S = 64.0
M2 = 0.5
THETA = math.cos(math.pi - M2)
COS_M2 = math.cos(M2)
SIN_M2 = math.sin(M2)
EPS = 1e-07

NC = 2
NS = 16
LANES = 16

BR = 1024
BC = 2048


def _sqrt16(a):
    seed_bits = jnp.int32(0x1FBD1DF5) + (lax.bitcast_convert_type(a, jnp.int32) >> 1)
    y = lax.bitcast_convert_type(seed_bits, jnp.float32)
    for _ in range(3):
        y = 0.5 * (y + a / y)
    return y


def _sc_body(n_cols, labels_hbm, logits_hbm, out_hbm, lab_v, idx_v, val_v, res_v, sem):
    wid = lax.axis_index("s") * NC + lax.axis_index("c")
    bpw = 2 * LANES
    base = wid * bpw
    pltpu.sync_copy(labels_hbm.at[pl.ds(base, bpw)], lab_v)
    for j in range(2):
        lab16 = lab_v[pl.ds(j * LANES, LANES)]
        rows = lax.iota(jnp.int32, LANES) + (base + j * LANES)
        idx_v[pl.ds(j * LANES, LANES)] = rows * n_cols + lab16
    pltpu.async_copy(logits_hbm.at[idx_v], val_v, sem).wait()
    for j in range(2):
        x = val_v[pl.ds(j * LANES, LANES)]
        tlc = jnp.clip(x, -1.0 + EPS, 1.0 - EPS)
        root = _sqrt16(jnp.maximum(1.0 - tlc * tlc, 1e-30))
        margined = tlc * COS_M2 - root * SIN_M2
        newv = jnp.where(x > THETA, margined, tlc)
        res_v[pl.ds(j * LANES, LANES)] = newv * S
    pltpu.sync_copy(res_v, out_hbm.at[pl.ds(base, bpw)])


def _sc_new_targets(labels, logits_flat, n_cols):
    n_rows = labels.shape[0]
    mesh = plsc.VectorSubcoreMesh(core_axis_name="c", subcore_axis_name="s")
    return pl.kernel(
        functools.partial(_sc_body, n_cols),
        out_type=jax.ShapeDtypeStruct((n_rows,), jnp.float32),
        mesh=mesh,
        scratch_types=[
            pltpu.VMEM((32,), jnp.int32),
            pltpu.VMEM((32,), jnp.int32),
            pltpu.VMEM((32,), jnp.float32),
            pltpu.VMEM((32,), jnp.float32),
            pltpu.SemaphoreType.DMA,
        ],
    )(labels, logits_flat)


def _tc_body(lab_ref, new_ref, x_ref, o_ref):
    j = pl.program_id(1)
    lab = lab_ref[...]
    nv = new_ref[...]
    local = lab - j * BC
    col_ids = jax.lax.broadcasted_iota(jnp.int32, (BR, BC), 1)
    mask = col_ids == local
    o_ref[...] = jnp.where(mask, nv, x_ref[...] * S)


def kernel(logits, labels):
    n_rows, n_cols = logits.shape
    newv = _sc_new_targets(labels, logits.reshape(-1), n_cols)
    grid = (n_rows // BR, pl.cdiv(n_cols, BC))
    return pl.pallas_call(
        _tc_body,
        grid=grid,
        in_specs=[
            pl.BlockSpec((BR, 1), lambda i, j: (i, 0)),
            pl.BlockSpec((BR, 1), lambda i, j: (i, 0)),
            pl.BlockSpec((BR, BC), lambda i, j: (i, j)),
        ],
        out_specs=pl.BlockSpec((BR, BC), lambda i, j: (i, j)),
        out_shape=jax.ShapeDtypeStruct((n_rows, n_cols), logits.dtype),
        compiler_params=pltpu.CompilerParams(
            dimension_semantics=("parallel", "parallel"),
        ),
    )(labels.reshape(n_rows, 1), newv.reshape(n_rows, 1), logits)


# speedup vs baseline: 1.6111x; 1.5833x over previous
import math

import jax
import jax.numpy as jnp
from jax.experimental import pallas as pl
from jax.experimental.pallas import tpu as pltpu

S = 64.0
M2 = 0.5
THETA = math.cos(math.pi - M2)
COS_M2 = math.cos(M2)
SIN_M2 = math.sin(M2)
EPS = 1e-07

BR = 256
BC = 4096


def _body(lab_ref, x_ref, o_ref):
    j = pl.program_id(1)
    x = x_ref[...]
    lab = lab_ref[...]
    valid = lab != -1
    lab0 = jnp.where(valid, lab, 0)
    local = lab0 - j * BC
    col_ids = jax.lax.broadcasted_iota(jnp.int32, (BR, BC), 1)
    mask = (col_ids == local) & valid
    tl = jnp.sum(jnp.where(mask, x, 0.0), axis=1, keepdims=True)
    tlc = jnp.clip(tl, -1.0 + EPS, 1.0 - EPS)
    margined = tlc * COS_M2 - jnp.sqrt(jnp.maximum(1.0 - tlc * tlc, 0.0)) * SIN_M2
    newv = jnp.where(tl > THETA, margined, tlc)
    o_ref[...] = jnp.where(mask, newv, x) * S


def kernel(logits, labels):
    n_rows, n_cols = logits.shape
    labels_2d = labels.reshape(n_rows, 1)
    grid = (n_rows // BR, pl.cdiv(n_cols, BC))
    return pl.pallas_call(
        _body,
        grid=grid,
        in_specs=[
            pl.BlockSpec((BR, 1), lambda i, j: (i, 0)),
            pl.BlockSpec((BR, BC), lambda i, j: (i, j)),
        ],
        out_specs=pl.BlockSpec((BR, BC), lambda i, j: (i, j)),
        out_shape=jax.ShapeDtypeStruct((n_rows, n_cols), logits.dtype),
        compiler_params=pltpu.CompilerParams(
            dimension_semantics=("parallel", "parallel"),
        ),
    )(labels_2d, logits)
